# baseline (speedup 1.0000x reference)
import jax
import jax.numpy as jnp
from jax import lax
from jax.experimental import pallas as pl
from jax.experimental.pallas import tpu as pltpu
from jax.experimental.pallas import tpu_sc as plsc

B = 4096
F = 32
N = 1000000
TAIL64 = (N // 128) * 128
LAST_TILE = (N - 128) // 128 * 128
RING = 8
NC = 2
NS = 16
NW = NC * NS
BPW = B // NW
CH = BPW // 16


def _mf_body(users_hbm, items_hbm, uemb_hbm, iemb_hbm, ub_hbm, ib_hbm,
             bias_hbm, out_hbm,
             uidx_v, iidx_v, ue_tiles, ie_tiles, ub_tiles, ib_tiles,
             ue_tail_v, ie_tail_v, ub_tail_v, ib_tail_v,
             ue_cols, ie_cols, ub_cols, ib_cols, ucol_buf, icol_buf,
             bias_v, out_v, sems):
    wid = lax.axis_index("s") * NC + lax.axis_index("c")
    base = wid * BPW

    pltpu.sync_copy(users_hbm.at[pl.ds(base, BPW)], uidx_v)
    pltpu.sync_copy(items_hbm.at[pl.ds(base, BPW)], iidx_v)
    pltpu.sync_copy(bias_hbm, bias_v.at[pl.ds(0, 1)])
    pltpu.sync_copy(uemb_hbm.at[:, pl.ds(TAIL64, 64)], ue_tail_v)
    pltpu.sync_copy(iemb_hbm.at[:, pl.ds(TAIL64, 64)], ie_tail_v)
    pltpu.sync_copy(ub_hbm.at[:, pl.ds(TAIL64, 64)], ub_tail_v)
    pltpu.sync_copy(ib_hbm.at[:, pl.ds(TAIL64, 64)], ib_tail_v)

    lanes = lax.iota(jnp.int32, 16)
    zeros16 = jnp.zeros((16,), jnp.int32)
    lane0 = lanes == 0

    def extract_scalar(idx_v, j):
        chunk = idx_v[pl.ds((j // 16) * 16, 16)]
        return jnp.sum(jnp.where(lanes == (j % 16), chunk, 0))

    def fire(j, r):
        u = extract_scalar(uidx_v, j)
        t = extract_scalar(iidx_v, j)
        su = pl.multiple_of(jnp.minimum((u // 128) * 128, LAST_TILE), 128)
        st = pl.multiple_of(jnp.minimum((t // 128) * 128, LAST_TILE), 128)
        pltpu.async_copy(uemb_hbm.at[:, pl.ds(su, 128)], ue_tiles.at[r],
                         sems.at[r])
        pltpu.async_copy(ub_hbm.at[:, pl.ds(su, 128)], ub_tiles.at[r],
                         sems.at[r])
        pltpu.async_copy(iemb_hbm.at[:, pl.ds(st, 128)], ie_tiles.at[r],
                         sems.at[r])
        pltpu.async_copy(ib_hbm.at[:, pl.ds(st, 128)], ib_tiles.at[r],
                         sems.at[r])
        j16 = jnp.broadcast_to(j, (16,)).astype(jnp.int32)
        plsc.store_scatter(ucol_buf, [j16],
                           jnp.broadcast_to(u - su, (16,)).astype(jnp.int32),
                           mask=lane0)
        plsc.store_scatter(icol_buf, [j16],
                           jnp.broadcast_to(t - st, (16,)).astype(jnp.int32),
                           mask=lane0)

    def drain(r):
        pltpu.make_async_copy(uemb_hbm.at[:, pl.ds(0, 128)],
                              ue_tiles.at[r], sems.at[r]).wait()
        pltpu.make_async_copy(ub_hbm.at[:, pl.ds(0, 128)],
                              ub_tiles.at[r], sems.at[r]).wait()
        pltpu.make_async_copy(iemb_hbm.at[:, pl.ds(0, 128)],
                              ie_tiles.at[r], sems.at[r]).wait()
        pltpu.make_async_copy(ib_hbm.at[:, pl.ds(0, 128)],
                              ib_tiles.at[r], sems.at[r]).wait()

    def extract(j, r, col_buf, tiles, b_tiles, tail_v, tail_b_v, cols_v,
                bcols_v):
        j16 = jnp.broadcast_to(j, (16,)).astype(jnp.int32)
        col = plsc.load_gather(col_buf, [j16])
        sel = col < 128
        cm = jnp.minimum(col, 127)
        ct = jnp.clip(col - 128, 0, 63)
        lo = plsc.load_gather(tiles.at[r], [lanes, cm])
        hi = plsc.load_gather(tiles.at[r], [lanes + 16, cm])
        lo_t = plsc.load_gather(tail_v, [lanes, ct])
        hi_t = plsc.load_gather(tail_v, [lanes + 16, ct])
        plsc.store_scatter(cols_v, [lanes, j16], jnp.where(sel, lo, lo_t))
        plsc.store_scatter(cols_v, [lanes + 16, j16], jnp.where(sel, hi, hi_t))
        bv = plsc.load_gather(b_tiles.at[r], [zeros16, cm])
        bv_t = plsc.load_gather(tail_b_v, [zeros16, ct])
        plsc.store_scatter(bcols_v, [j16], jnp.where(sel, bv, bv_t),
                           mask=lane0)

    for r in range(RING):
        fire(r, r)

    def body(g, carry):
        for r in range(RING):
            j = g * RING + r
            drain(r)
            extract(j, r, ucol_buf, ue_tiles, ub_tiles, ue_tail_v, ub_tail_v,
                    ue_cols, ub_cols)
            extract(j, r, icol_buf, ie_tiles, ib_tiles, ie_tail_v, ib_tail_v,
                    ie_cols, ib_cols)
            jn = j + RING

            @pl.when(jn < BPW)
            def _():
                fire(jn, r)
        return carry

    lax.fori_loop(0, BPW // RING, body, 0)

    bias_s = jnp.sum(jnp.where(lane0, bias_v[...], 0.0))
    bias_vec = jnp.broadcast_to(bias_s, (16,))
    for c in range(CH):
        sl = pl.ds(c * 16, 16)
        acc = ub_cols[sl] + ib_cols[sl] + bias_vec
        for f in range(F):
            acc = acc + ue_cols[f, sl] * ie_cols[f, sl]
        out_v[sl] = jnp.clip(acc, 1.0, 5.0)

    pltpu.sync_copy(out_v, out_hbm.at[pl.ds(base, BPW)])


def kernel(users, items, user_embeddings, item_embeddings, user_biases,
           item_biases, bias):
    uemb_t = user_embeddings.T
    iemb_t = item_embeddings.T
    ub_t = user_biases.T
    ib_t = item_biases.T
    mesh = plsc.VectorSubcoreMesh(core_axis_name="c", subcore_axis_name="s")
    run = pl.kernel(
        _mf_body,
        mesh=mesh,
        compiler_params=pltpu.CompilerParams(needs_layout_passes=False),
        out_type=jax.ShapeDtypeStruct((B,), jnp.float32),
        scratch_types=[
            pltpu.VMEM((BPW,), jnp.int32),
            pltpu.VMEM((BPW,), jnp.int32),
            pltpu.VMEM((RING, F, 128), jnp.float32),
            pltpu.VMEM((RING, F, 128), jnp.float32),
            pltpu.VMEM((RING, 1, 128), jnp.float32),
            pltpu.VMEM((RING, 1, 128), jnp.float32),
            pltpu.VMEM((F, 64), jnp.float32),
            pltpu.VMEM((F, 64), jnp.float32),
            pltpu.VMEM((1, 64), jnp.float32),
            pltpu.VMEM((1, 64), jnp.float32),
            pltpu.VMEM((F, BPW), jnp.float32),
            pltpu.VMEM((F, BPW), jnp.float32),
            pltpu.VMEM((BPW,), jnp.float32),
            pltpu.VMEM((BPW,), jnp.float32),
            pltpu.VMEM((BPW,), jnp.int32),
            pltpu.VMEM((BPW,), jnp.int32),
            pltpu.VMEM((16,), jnp.float32),
            pltpu.VMEM((BPW,), jnp.float32),
            pltpu.SemaphoreType.DMA((RING,)),
        ],
    )
    return run(users, items, uemb_t, iemb_t, ub_t, ib_t, bias)
